# baseline (speedup 1.0000x reference)
import functools

import jax
import jax.numpy as jnp
from jax import lax
from jax.experimental import pallas as pl
from jax.experimental.pallas import tpu as pltpu
from jax.experimental.pallas import tpu_sc as plsc

_N = 10000
_E = 320000
_H = 8
_C = 16
_HC = 128
_HW = _HC + 16
_G = 64

_NPAD = 10112
_NC = 2
_NS = 16
_NW = _NC * _NS
_EPW = 10240
_BB = 64
_NBLK = _EPW // _BB
_GB = 8
_RPS = _NPAD // _NS

_RB = 128
_NRB = _NPAD // _RB

f32 = jnp.float32
i32 = jnp.int32


def _dense_body(x_ref, waug_ref, wd_ref, hs_ref, d_ref):
    x = x_ref[...]
    hs_ref[...] = jnp.dot(x, waug_ref[...], preferred_element_type=f32)
    d_ref[...] = jnp.dot(x, wd_ref[...], preferred_element_type=f32)


_dense_call = pl.pallas_call(
    _dense_body,
    grid=(_NRB,),
    in_specs=[
        pl.BlockSpec((_RB, _HC), lambda i: (i, 0)),
        pl.BlockSpec((_HC, _HW), lambda i: (0, 0)),
        pl.BlockSpec((_HC, 16), lambda i: (0, 0)),
    ],
    out_specs=[
        pl.BlockSpec((_RB, _HW), lambda i: (i, 0)),
        pl.BlockSpec((_RB, 16), lambda i: (i, 0)),
    ],
    out_shape=[
        jax.ShapeDtypeStruct((_NPAD, _HW), f32),
        jax.ShapeDtypeStruct((_NPAD, 16), f32),
    ],
)


def _mid_body(pa_ref, bias_ref, exp_ref, waug_ref, wd_ref,
              hs_ref, d_ref):
    acc = pa_ref[0, :, :_HC] + pa_ref[1, :, :_HC]
    den16 = pa_ref[0, :, _HC:] + pa_ref[1, :, _HC:]
    den = jnp.dot(den16, exp_ref[...], preferred_element_type=f32)
    x_in = acc / (den + 1e-16) + bias_ref[...]
    x_in = jnp.maximum(x_in, 0.0)
    hs_ref[...] = jnp.dot(x_in, waug_ref[...], preferred_element_type=f32)
    d_ref[...] = jnp.dot(x_in, wd_ref[...], preferred_element_type=f32)


_mid_call = pl.pallas_call(
    _mid_body,
    grid=(_NRB,),
    in_specs=[
        pl.BlockSpec((_NC, _RB, _HW), lambda i: (0, i, 0)),
        pl.BlockSpec((1, _HC), lambda i: (0, 0)),
        pl.BlockSpec((16, _HC), lambda i: (0, 0)),
        pl.BlockSpec((_HC, _HW), lambda i: (0, 0)),
        pl.BlockSpec((_HC, 16), lambda i: (0, 0)),
    ],
    out_specs=[
        pl.BlockSpec((_RB, _HW), lambda i: (i, 0)),
        pl.BlockSpec((_RB, 16), lambda i: (i, 0)),
    ],
    out_shape=[
        jax.ShapeDtypeStruct((_NPAD, _HW), f32),
        jax.ShapeDtypeStruct((_NPAD, 16), f32),
    ],
)


def _pool_body(pa_ref, bias_ref, exp_ref, batch_ref, fcw_ref, fcb_ref,
               out_ref, sums_ref, cnt_ref):
    i = pl.program_id(0)

    @pl.when(i == 0)
    def _():
        sums_ref[...] = jnp.zeros_like(sums_ref)
        cnt_ref[...] = jnp.zeros_like(cnt_ref)

    acc = pa_ref[0, :, :_HC] + pa_ref[1, :, :_HC]
    den16 = pa_ref[0, :, _HC:] + pa_ref[1, :, _HC:]
    den = jnp.dot(den16, exp_ref[...], preferred_element_type=f32)
    node = acc / (den + 1e-16) + bias_ref[...]
    b_ids = batch_ref[0]
    gids = lax.broadcasted_iota(i32, (_G, _RB), 0)
    mask = (b_ids == gids).astype(f32)
    sums_ref[...] += jnp.dot(mask, node, preferred_element_type=f32)
    cnt_ref[...] += jnp.sum(mask, axis=1, keepdims=True)

    @pl.when(i == _NRB - 1)
    def _():
        pooled = sums_ref[...] / jnp.maximum(cnt_ref[...], 1.0)
        out_ref[...] = (jnp.sum(pooled * fcw_ref[...], axis=1, keepdims=True)
                        + fcb_ref[0, 0])


_pool_call = pl.pallas_call(
    _pool_body,
    grid=(_NRB,),
    in_specs=[
        pl.BlockSpec((_NC, _RB, _HW), lambda i: (0, i, 0)),
        pl.BlockSpec((1, _HC), lambda i: (0, 0)),
        pl.BlockSpec((16, _HC), lambda i: (0, 0)),
        pl.BlockSpec((1, 1, _RB), lambda i: (i, 0, 0)),
        pl.BlockSpec((1, _HC), lambda i: (0, 0)),
        pl.BlockSpec((1, 1), lambda i: (0, 0)),
    ],
    out_specs=pl.BlockSpec((_G, 1), lambda i: (0, 0)),
    out_shape=jax.ShapeDtypeStruct((_G, 1), f32),
    scratch_shapes=[
        pltpu.VMEM((_G, _HC), f32),
        pltpu.VMEM((_G, 1), f32),
    ],
)


@functools.lru_cache(maxsize=None)
def _make_edge_pass():
  mesh = plsc.VectorSubcoreMesh(core_axis_name="c", subcore_axis_name="s",
                                num_cores=_NC, num_subcores=_NS)

  ring_buf = [
      pltpu.VMEM((_BB,), i32),
      pltpu.VMEM((_BB,), i32),
      pltpu.VMEM((_BB, _HW), f32),
      pltpu.VMEM((_BB, 16), f32),
      pltpu.SemaphoreType.DMA,
      pltpu.SemaphoreType.DMA,
      pltpu.SemaphoreType.DMA,
  ]

  @functools.partial(
    pl.kernel,
    out_type=jax.ShapeDtypeStruct((_NC, _NPAD, _HW), f32),
    mesh=mesh,
    compiler_params=pltpu.CompilerParams(use_tc_tiling_on_sc=False),
    scratch_types=[
        pltpu.VMEM((_GB, 2 * _BB), i32),
        pltpu.VMEM_SHARED((_NPAD, _HW), f32),
    ] + ring_buf * 3,
  )
  def _edge_pass(edges_h, hs_h, d_h, acc_o, edgeg, accs, *ring):
    bufs = [ring[7 * k:7 * (k + 1)] for k in range(3)]
    cid = lax.axis_index("c")
    sid = lax.axis_index("s")
    wid = sid * _NC + cid

    hr0 = bufs[0][2]
    zv = jnp.zeros((16,), f32)

    def zi(k, _):
        for j in range(_HW // 16):
            hr0[k, pl.ds(16 * j, 16)] = zv
        return 0

    lax.fori_loop(0, _BB, zi, 0)
    r0 = sid * _RPS
    for t in range(_RPS // _BB):
        pltpu.sync_copy(hr0, accs.at[pl.ds(r0 + t * _BB, _BB)])
    rem = _RPS - (_RPS // _BB) * _BB
    pltpu.sync_copy(hr0.at[pl.ds(0, rem)],
                    accs.at[pl.ds(r0 + _RPS - rem, rem)])
    plsc.subcore_barrier()

    base_blk = wid * _NBLK
    nblk = jnp.minimum(_NBLK, jnp.maximum(0, (_E - wid * _EPW) // _BB))

    def load_group(first_blk):
        g = pl.multiple_of(base_blk + first_blk, _GB)
        pltpu.sync_copy(edges_h.at[pl.ds(g, _GB)], edgeg)

    def stage(j, k):
        sv, dv = bufs[k][0], bufs[k][1]
        row = lax.rem(j, _GB)
        for m in range(_BB // 16):
            sv[pl.ds(16 * m, 16)] = edgeg[row, pl.ds(16 * m, 16)]
            dv[pl.ds(16 * m, 16)] = edgeg[row, pl.ds(_BB + 16 * m, 16)]
        hr, dr = bufs[k][2], bufs[k][3]
        g_d, g_h = bufs[k][4], bufs[k][5]
        pltpu.async_copy(d_h.at[dv], dr, g_d)
        pltpu.async_copy(hs_h.at[sv], hr, g_h)

    def wait_gathers(k):
        sv, dv, hr, dr = bufs[k][0], bufs[k][1], bufs[k][2], bufs[k][3]
        g_d, g_h = bufs[k][4], bufs[k][5]
        pltpu.make_async_copy(d_h.at[dv], dr, g_d).wait()
        pltpu.make_async_copy(hs_h.at[sv], hr, g_h).wait()

    def issue_scatter(k):
        dv, hr, g_sc = bufs[k][1], bufs[k][2], bufs[k][6]
        pltpu.async_copy(hr, accs.at[dv], g_sc, add=True)

    def wait_scatter(k):
        dv, hr, g_sc = bufs[k][1], bufs[k][2], bufs[k][6]
        pltpu.make_async_copy(hr, accs.at[dv], g_sc).wait()

    def compute(k):
        hr, dr = bufs[k][2], bufs[k][3]

        def edge_w(b, _):
            e = hr[b, pl.ds(_HC, 16)] + dr[b, :]
            e = jnp.where(e > 0, e, f32(0.2) * e)
            hr[b, pl.ds(_HC, 16)] = jnp.exp(e)
            return 0

        lax.fori_loop(0, _BB, edge_w, 0)

        def edge_m(b, _):
            w = hr[b, pl.ds(_HC, 16)]
            for j in range(_H):
                hr[b, pl.ds(16 * j, 16)] = hr[b, pl.ds(16 * j, 16)] * w[j]
            return 0

        lax.fori_loop(0, _BB, edge_m, 0)

    load_group(0)
    stage(0, 0)
    stage(1, 1)

    def step(i3, _):
        for k in range(3):
            j = 3 * i3 + k

            @pl.when(j < nblk)
            def _():
                k2 = (k + 2) % 3
                wait_gathers(k)
                compute(k)

                @pl.when(j + 2 < nblk)
                def _():
                    @pl.when(j >= 1)
                    def _():
                        wait_scatter(k2)

                    @pl.when(lax.rem(j + 2, _GB) == 0)
                    def _():
                        load_group(j + 2)

                    stage(j + 2, k2)

                issue_scatter(k)

        return 0

    lax.fori_loop(0, (nblk + 2) // 3, step, 0)
    for k in range(3):
        wait_scatter(k)
    plsc.subcore_barrier()
    pltpu.sync_copy(accs.at[pl.ds(r0, _RPS)], acc_o.at[cid, pl.ds(r0, _RPS)])

  return _edge_pass


def _attn_mats(a_src, a_dst):
    eye = jnp.eye(_H, dtype=f32)
    a_s = a_src.reshape(_H, _C)
    a_d = a_dst.reshape(_H, _C)
    As8 = (a_s[:, :, None] * eye[:, None, :]).reshape(_HC, _H)
    Ad8 = (a_d[:, :, None] * eye[:, None, :]).reshape(_HC, _H)
    return (jnp.concatenate([As8, As8], axis=1),
            jnp.concatenate([Ad8, Ad8], axis=1))


def _expand_mat():
    e8 = jnp.kron(jnp.eye(_H, dtype=f32), jnp.ones((1, _C), f32)) * 0.5
    return jnp.concatenate([e8, e8], axis=0)


def kernel(x, edge_index, batch, W1, a_src1, a_dst1, b1, W2, a_src2, a_dst2,
           b2, W3, a_src3, a_dst3, b3, fc_w, fc_b):
    edges2d = jnp.concatenate(
        [edge_index[0].reshape(_E // _BB, _BB),
         edge_index[1].reshape(_E // _BB, _BB)], axis=1)
    xp = jnp.pad(x, ((0, _NPAD - _N), (0, 0)))
    expand = _expand_mat()
    batch3d = jnp.pad(batch, (0, _NPAD - _N), constant_values=_G).reshape(
        _NRB, 1, _RB)

    As1, Ad1 = _attn_mats(a_src1, a_dst1)
    As2, Ad2 = _attn_mats(a_src2, a_dst2)
    As3, Ad3 = _attn_mats(a_src3, a_dst3)
    edge_pass = _make_edge_pass()

    def aug(W, As):
        return jnp.concatenate([W, jnp.dot(W, As)], axis=1)

    Waugs = jnp.stack([aug(W1, As1), aug(W2, As2), aug(W3, As3)])
    Wds = jnp.stack([jnp.dot(W1, Ad1), jnp.dot(W2, Ad2), jnp.dot(W3, Ad3)])
    bs = jnp.stack([jnp.zeros_like(b1), b1, b2]).reshape(3, 1, _HC)

    def body(pa, xs):
        Waugl, Wdl, bl, first = xs
        hs, d = lax.cond(
            first,
            lambda: _dense_call(xp, Waugl, Wdl),
            lambda: _mid_call(pa, bl, expand, Waugl, Wdl),
        )
        pa2 = edge_pass(edges2d, hs, d)
        return pa2, None

    pa0 = jnp.zeros((_NC, _NPAD, _HW), f32)
    pa, _ = lax.scan(
        body, pa0, (Waugs, Wds, bs, jnp.array([True, False, False])))
    out = _pool_call(pa, b3.reshape(1, _HC), expand, batch3d,
                     fc_w.reshape(1, _HC), fc_b.reshape(1, 1))
    return out
